# baseline (speedup 1.0000x reference)
import jax
import jax.numpy as jnp
from jax import lax
from jax.experimental import pallas as pl
from jax.experimental.pallas import tpu as pltpu
from jax.experimental.pallas import tpu_sc as plsc

_N = 10000
_E = 320000
_F_IN = 128
_HID = 64
_HID2 = 32
_N_DCS = 32
_N_GRAPHS = 16

_NC = 2
_NS = 16
_NW = _NC * _NS
_K = 256
_C = 40
_CS = 20
_CF = 2 * _C - _CS
_E_PAD = _NW * _C * _K
_N_PAD = 10240
_RPT = _N_PAD // _NS
_DEG_W = 16

_BR = 400
_G = _N // _BR

_SC_PARAMS = pltpu.CompilerParams(use_tc_tiling_on_sc=False)


def _sc_mesh():
  return plsc.VectorSubcoreMesh(core_axis_name="c", subcore_axis_name="s")


def _sc_deg():
  def body(dstm, ones_rows, zrows, out, didx, obuf, acc, sem):
    cid = lax.axis_index("c")
    sid = lax.axis_index("s")
    myc = jnp.where(cid == 0, _CF, _CS)
    base = (cid * _NS + sid) * _CF
    pltpu.sync_copy(dstm.at[pl.ds(base, _CF)], didx)
    pltpu.sync_copy(ones_rows, obuf)
    pltpu.sync_copy(zrows, acc.at[pl.ds(sid * _RPT, _RPT)])
    plsc.subcore_barrier()

    def group(g, carry):
      cps = []
      for b in range(4):
        j = g * 4 + b
        cps.append(pltpu.async_copy(obuf, acc.at[didx.at[j]], sem, add=True))
      for cp in cps:
        cp.wait()
      return carry

    lax.fori_loop(0, myc // 4, group, 0)
    plsc.subcore_barrier()
    pltpu.sync_copy(acc.at[pl.ds(sid * _RPT, _RPT)],
                    out.at[cid, pl.ds(sid * _RPT, _RPT)])

  return pl.kernel(
      body,
      out_type=jax.ShapeDtypeStruct((_NC, _N_PAD, _DEG_W), jnp.float32),
      mesh=_sc_mesh(),
      scratch_types=[
          pltpu.VMEM((_CF, _K), jnp.int32),
          pltpu.VMEM((_K, _DEG_W), jnp.float32),
          pltpu.VMEM_SHARED((_N_PAD, _DEG_W), jnp.float32),
          pltpu.SemaphoreType.DMA,
      ],
      compiler_params=_SC_PARAMS,
  )


def _sc_agg(d):
  nbuf = 2

  def body(table, srcm, dstm, zrows, out, sidx, didx, rows, acc, *sems):
    gsems = sems[:nbuf]
    ssems = sems[nbuf:]
    cid = lax.axis_index("c")
    sid = lax.axis_index("s")
    myc = jnp.where(cid == 0, _CF, _CS)
    base = (cid * _NS + sid) * _CF
    pltpu.sync_copy(srcm.at[pl.ds(base, _CF)], sidx)
    pltpu.sync_copy(dstm.at[pl.ds(base, _CF)], didx)
    pltpu.sync_copy(zrows, acc.at[pl.ds(sid * _RPT, _RPT)])
    plsc.subcore_barrier()

    for b in range(nbuf - 1):
      pltpu.async_copy(table.at[sidx.at[b]], rows.at[b], gsems[b])

    def group(g, carry):
      for u in range(nbuf):
        j = g * nbuf + u
        b = u
        pltpu.make_async_copy(table.at[sidx.at[j]], rows.at[b],
                              gsems[b]).wait()
        pltpu.async_copy(rows.at[b], acc.at[didx.at[j]], ssems[b], add=True)
        nj = j + nbuf - 1
        bp = (u + nbuf - 1) % nbuf

        @pl.when(nj < myc)
        def _():
          @pl.when(j > 0)
          def _():
            pltpu.make_async_copy(rows.at[bp], acc.at[didx.at[j]],
                                  ssems[bp]).wait()

          pltpu.async_copy(table.at[sidx.at[nj]], rows.at[bp], gsems[bp])
      return carry

    lax.fori_loop(0, myc // nbuf, group, 0)
    for b in range(nbuf):
      pltpu.make_async_copy(rows.at[b], acc.at[didx.at[b]], ssems[b]).wait()
    plsc.subcore_barrier()
    pltpu.sync_copy(acc.at[pl.ds(sid * _RPT, _RPT)],
                    out.at[cid, pl.ds(sid * _RPT, _RPT)])

  return pl.kernel(
      body,
      out_type=jax.ShapeDtypeStruct((_NC, _N_PAD, d), jnp.float32),
      mesh=_sc_mesh(),
      scratch_types=[
          pltpu.VMEM((_CF, _K), jnp.int32),
          pltpu.VMEM((_CF, _K), jnp.int32),
          pltpu.VMEM((nbuf, _K, d), jnp.float32),
          pltpu.VMEM_SHARED((_N_PAD, d), jnp.float32),
      ] + [pltpu.SemaphoreType.DMA] * (2 * nbuf),
      compiler_params=_SC_PARAMS,
  )


def _tc1_body(counts_ref, x_ref, w1_ref, h_ref, dis_ref):
  counts = counts_ref[...]
  deg = counts[0, :, :1] + counts[1, :, :1] + 1.0
  dis = lax.rsqrt(deg)
  h = jnp.dot(x_ref[...], w1_ref[...], preferred_element_type=jnp.float32)
  h_ref[...] = h * dis
  dis_ref[...] = dis


def _tc1(counts, x, w1):
  return pl.pallas_call(
      _tc1_body,
      grid=(_G,),
      in_specs=[
          pl.BlockSpec((_NC, _BR, _DEG_W), lambda i: (0, i, 0)),
          pl.BlockSpec((_BR, _F_IN), lambda i: (i, 0)),
          pl.BlockSpec((_F_IN, _HID), lambda i: (0, 0)),
      ],
      out_specs=[
          pl.BlockSpec((_BR, _HID), lambda i: (i, 0)),
          pl.BlockSpec((_BR, 1), lambda i: (i, 0)),
      ],
      out_shape=[
          jax.ShapeDtypeStruct((_N, _HID), jnp.float32),
          jax.ShapeDtypeStruct((_N, 1), jnp.float32),
      ],
  )(counts, x, w1)


def _tc2_body(p_ref, h1s_ref, dis_ref, b1_ref, w2_ref, o_ref):
  p = p_ref[...]
  dis = dis_ref[...]
  s = p[0] + p[1] + h1s_ref[...]
  bx = jnp.maximum(dis * s + b1_ref[...], 0.0)
  o_ref[...] = jnp.dot(bx, w2_ref[...],
                       preferred_element_type=jnp.float32) * dis


def _tc2(p1, h1s, dis, b1, w2):
  return pl.pallas_call(
      _tc2_body,
      grid=(_G,),
      in_specs=[
          pl.BlockSpec((_NC, _BR, _HID), lambda i: (0, i, 0)),
          pl.BlockSpec((_BR, _HID), lambda i: (i, 0)),
          pl.BlockSpec((_BR, 1), lambda i: (i, 0)),
          pl.BlockSpec((1, _HID), lambda i: (0, 0)),
          pl.BlockSpec((_HID, _HID2), lambda i: (0, 0)),
      ],
      out_specs=pl.BlockSpec((_BR, _HID2), lambda i: (i, 0)),
      out_shape=jax.ShapeDtypeStruct((_N, _HID2), jnp.float32),
  )(p1, h1s, dis, b1, w2)


def _tc3_body(p_ref, h2s_ref, dis_ref, b2_ref, ids_ref, wm_ref, bm_ref,
              o_ref, px_acc):
  i = pl.program_id(0)

  @pl.when(i == 0)
  def _():
    px_acc[...] = jnp.full((_N_GRAPHS, _HID2), -jnp.inf, jnp.float32)

  p = p_ref[...]
  cx = jnp.maximum(dis_ref[...] * (p[0] + p[1] + h2s_ref[...])
                   + b2_ref[...], 0.0)
  gids = lax.broadcasted_iota(jnp.int32, (_N_GRAPHS, _BR, 1), 0)
  m = gids == ids_ref[...][None, :, :]
  vals = jnp.where(m, cx[None, :, :], -jnp.inf)
  px_acc[...] = jnp.maximum(px_acc[...], jnp.max(vals, axis=1))

  @pl.when(i == _G - 1)
  def _():
    o_ref[...] = jnp.dot(px_acc[...], wm_ref[...],
                         preferred_element_type=jnp.float32) + bm_ref[...]


def _tc3(p2, h2s, dis, b2, ids, wm, bm):
  return pl.pallas_call(
      _tc3_body,
      grid=(_G,),
      in_specs=[
          pl.BlockSpec((_NC, _BR, _HID2), lambda i: (0, i, 0)),
          pl.BlockSpec((_BR, _HID2), lambda i: (i, 0)),
          pl.BlockSpec((_BR, 1), lambda i: (i, 0)),
          pl.BlockSpec((1, _HID2), lambda i: (0, 0)),
          pl.BlockSpec((_BR, 1), lambda i: (i, 0)),
          pl.BlockSpec((_HID2, _N_DCS), lambda i: (0, 0)),
          pl.BlockSpec((1, _N_DCS), lambda i: (0, 0)),
      ],
      out_specs=pl.BlockSpec((_N_GRAPHS, _N_DCS), lambda i: (0, 0)),
      out_shape=jax.ShapeDtypeStruct((_N_GRAPHS, _N_DCS), jnp.float32),
      scratch_shapes=[pltpu.VMEM((_N_GRAPHS, _N_DCS), jnp.float32)],
  )(p2, h2s, dis, b2, ids, wm, bm)


def kernel(x, edge_index, batch, W1, b1, W2, b2, Wm, bm):
  src = edge_index[0]
  dst = edge_index[1]
  pad = _E_PAD - _E

  def _layout(flat, fill):
    flat = jnp.concatenate([flat, jnp.full((pad,), fill, jnp.int32)])
    p0 = flat[:_NS * _CF * _K].reshape(_NS, _CF, _K)
    p1 = flat[_NS * _CF * _K:].reshape(_NS, _CS, _K)
    p1 = jnp.concatenate(
        [p1, jnp.full((_NS, _CF - _CS, _K), fill, jnp.int32)], axis=1)
    return jnp.concatenate([p0, p1], axis=0).reshape(_NW * _CF, _K)

  srcm = _layout(src, 0)
  dstm = _layout(dst, _N)

  ones_rows = jnp.ones((_K, _DEG_W), jnp.float32)
  z_deg = jnp.zeros((_RPT, _DEG_W), jnp.float32)
  z1 = jnp.zeros((_RPT, _HID), jnp.float32)
  z2 = jnp.zeros((_RPT, _HID2), jnp.float32)

  counts = _sc_deg()(dstm, ones_rows, z_deg)
  h1s, dis = _tc1(counts, x, W1)
  p1 = _sc_agg(_HID)(h1s, srcm, dstm, z1)
  h2s = _tc2(p1, h1s, dis, b1.reshape(1, _HID), W2)
  p2 = _sc_agg(_HID2)(h2s, srcm, dstm, z2)
  return _tc3(p2, h2s, dis, b2.reshape(1, _HID2),
              batch.reshape(_N, 1), Wm, bm.reshape(1, _N_DCS))


# speedup vs baseline: 1.3441x; 1.3441x over previous
import jax
import jax.numpy as jnp
from jax import lax
from jax.experimental import pallas as pl
from jax.experimental.pallas import tpu as pltpu
from jax.experimental.pallas import tpu_sc as plsc

_N = 10000
_E = 320000
_F_IN = 128
_HID = 64
_HID2 = 32
_N_DCS = 32
_N_GRAPHS = 16

_NC = 2
_NS = 16
_NW = _NC * _NS
_K = 256
_C = 40
_CS = 20
_CF = 2 * _C - _CS
_E_PAD = _NW * _C * _K
_N_PAD = 10240
_RPT = _N_PAD // _NS
_DEG_W = 16

_BR = 400
_G = _N // _BR

_SC_PARAMS = pltpu.CompilerParams(use_tc_tiling_on_sc=False)


def _sc_mesh():
  return plsc.VectorSubcoreMesh(core_axis_name="c", subcore_axis_name="s")


def _sc_deg():
  def body(dstm, ones_rows, zrows, out, didx, obuf, acc, sem):
    cid = lax.axis_index("c")
    sid = lax.axis_index("s")
    myc = jnp.where(cid == 0, _CF, _CS)
    base = (cid * _NS + sid) * _CF
    pltpu.sync_copy(dstm.at[pl.ds(base, _CF)], didx)
    pltpu.sync_copy(ones_rows, obuf)
    pltpu.sync_copy(zrows, acc.at[pl.ds(sid * _RPT, _RPT)])
    plsc.subcore_barrier()

    def group(g, carry):
      cps = []
      for b in range(4):
        j = g * 4 + b
        cps.append(pltpu.async_copy(obuf, acc.at[didx.at[j]], sem, add=True))
      for cp in cps:
        cp.wait()
      return carry

    lax.fori_loop(0, myc // 4, group, 0)
    plsc.subcore_barrier()
    pltpu.sync_copy(acc.at[pl.ds(sid * _RPT, _RPT)],
                    out.at[cid, pl.ds(sid * _RPT, _RPT)])

  return pl.kernel(
      body,
      out_type=jax.ShapeDtypeStruct((_NC, _N_PAD, _DEG_W), jnp.float32),
      mesh=_sc_mesh(),
      scratch_types=[
          pltpu.VMEM((_CF, _K), jnp.int32),
          pltpu.VMEM((_K, _DEG_W), jnp.float32),
          pltpu.VMEM_SHARED((_N_PAD, _DEG_W), jnp.float32),
          pltpu.SemaphoreType.DMA,
      ],
      compiler_params=_SC_PARAMS,
  )


def _sc_agg(d):
  nbuf = 2

  def body(table, srcm, dstm, zrows, out, sidx, didx, rows, acc, *sems):
    gsems = sems[:nbuf]
    ssems = sems[nbuf:]
    cid = lax.axis_index("c")
    sid = lax.axis_index("s")
    myc = jnp.where(cid == 0, _CF, _CS)
    base = (cid * _NS + sid) * _CF
    pltpu.sync_copy(srcm.at[pl.ds(base, _CF)], sidx)
    pltpu.sync_copy(dstm.at[pl.ds(base, _CF)], didx)
    pltpu.sync_copy(zrows, acc.at[pl.ds(sid * _RPT, _RPT)])
    plsc.subcore_barrier()

    for b in range(nbuf - 1):
      pltpu.async_copy(table.at[sidx.at[b]], rows.at[b], gsems[b])

    def group(g, carry):
      for u in range(nbuf):
        j = g * nbuf + u
        b = u
        pltpu.make_async_copy(table.at[sidx.at[j]], rows.at[b],
                              gsems[b]).wait()
        pltpu.async_copy(rows.at[b], acc.at[didx.at[j]], ssems[b], add=True)
        nj = j + nbuf - 1
        bp = (u + nbuf - 1) % nbuf

        @pl.when(nj < myc)
        def _():
          @pl.when(j > 0)
          def _():
            pltpu.make_async_copy(rows.at[bp], acc.at[didx.at[j]],
                                  ssems[bp]).wait()

          pltpu.async_copy(table.at[sidx.at[nj]], rows.at[bp], gsems[bp])
      return carry

    lax.fori_loop(0, myc // nbuf, group, 0)
    for b in range(nbuf):
      pltpu.make_async_copy(rows.at[b], acc.at[didx.at[b]], ssems[b]).wait()
    plsc.subcore_barrier()
    pltpu.sync_copy(acc.at[pl.ds(sid * _RPT, _RPT)],
                    out.at[cid, pl.ds(sid * _RPT, _RPT)])

  return pl.kernel(
      body,
      out_type=jax.ShapeDtypeStruct((_NC, _N_PAD, d), jnp.bfloat16),
      mesh=_sc_mesh(),
      scratch_types=[
          pltpu.VMEM((_CF, _K), jnp.int32),
          pltpu.VMEM((_CF, _K), jnp.int32),
          pltpu.VMEM((nbuf, _K, d), jnp.bfloat16),
          pltpu.VMEM_SHARED((_N_PAD, d), jnp.bfloat16),
      ] + [pltpu.SemaphoreType.DMA] * (2 * nbuf),
      compiler_params=_SC_PARAMS,
  )


def _tc1_body(counts_ref, x_ref, w1_ref, h_ref, dis_ref):
  counts = counts_ref[...]
  deg = counts[0, :, :1] + counts[1, :, :1] + 1.0
  dis = lax.rsqrt(deg)
  h = jnp.dot(x_ref[...], w1_ref[...], preferred_element_type=jnp.float32)
  h_ref[...] = (h * dis).astype(jnp.bfloat16)
  dis_ref[...] = dis


def _tc1(counts, x, w1):
  return pl.pallas_call(
      _tc1_body,
      grid=(_G,),
      in_specs=[
          pl.BlockSpec((_NC, _BR, _DEG_W), lambda i: (0, i, 0)),
          pl.BlockSpec((_BR, _F_IN), lambda i: (i, 0)),
          pl.BlockSpec((_F_IN, _HID), lambda i: (0, 0)),
      ],
      out_specs=[
          pl.BlockSpec((_BR, _HID), lambda i: (i, 0)),
          pl.BlockSpec((_BR, 1), lambda i: (i, 0)),
      ],
      out_shape=[
          jax.ShapeDtypeStruct((_N, _HID), jnp.bfloat16),
          jax.ShapeDtypeStruct((_N, 1), jnp.float32),
      ],
  )(counts, x, w1)


def _tc2_body(p_ref, h1s_ref, dis_ref, b1_ref, w2_ref, o_ref):
  p = p_ref[...].astype(jnp.float32)
  dis = dis_ref[...]
  s = p[0] + p[1] + h1s_ref[...].astype(jnp.float32)
  bx = jnp.maximum(dis * s + b1_ref[...], 0.0)
  h2s = jnp.dot(bx, w2_ref[...], preferred_element_type=jnp.float32) * dis
  o_ref[...] = h2s.astype(jnp.bfloat16)


def _tc2(p1, h1s, dis, b1, w2):
  return pl.pallas_call(
      _tc2_body,
      grid=(_G,),
      in_specs=[
          pl.BlockSpec((_NC, _BR, _HID), lambda i: (0, i, 0)),
          pl.BlockSpec((_BR, _HID), lambda i: (i, 0)),
          pl.BlockSpec((_BR, 1), lambda i: (i, 0)),
          pl.BlockSpec((1, _HID), lambda i: (0, 0)),
          pl.BlockSpec((_HID, _HID2), lambda i: (0, 0)),
      ],
      out_specs=pl.BlockSpec((_BR, _HID2), lambda i: (i, 0)),
      out_shape=jax.ShapeDtypeStruct((_N, _HID2), jnp.bfloat16),
  )(p1, h1s, dis, b1, w2)


def _tc3_body(p_ref, h2s_ref, dis_ref, b2_ref, ids_ref, wm_ref, bm_ref,
              o_ref, px_acc):
  i = pl.program_id(0)

  @pl.when(i == 0)
  def _():
    px_acc[...] = jnp.full((_N_GRAPHS, _HID2), -jnp.inf, jnp.float32)

  p = p_ref[...].astype(jnp.float32)
  cx = jnp.maximum(dis_ref[...] * (p[0] + p[1]
                                   + h2s_ref[...].astype(jnp.float32))
                   + b2_ref[...], 0.0)
  gids = lax.broadcasted_iota(jnp.int32, (_N_GRAPHS, _BR, 1), 0)
  m = gids == ids_ref[...][None, :, :]
  vals = jnp.where(m, cx[None, :, :], -jnp.inf)
  px_acc[...] = jnp.maximum(px_acc[...], jnp.max(vals, axis=1))

  @pl.when(i == _G - 1)
  def _():
    o_ref[...] = jnp.dot(px_acc[...], wm_ref[...],
                         preferred_element_type=jnp.float32) + bm_ref[...]


def _tc3(p2, h2s, dis, b2, ids, wm, bm):
  return pl.pallas_call(
      _tc3_body,
      grid=(_G,),
      in_specs=[
          pl.BlockSpec((_NC, _BR, _HID2), lambda i: (0, i, 0)),
          pl.BlockSpec((_BR, _HID2), lambda i: (i, 0)),
          pl.BlockSpec((_BR, 1), lambda i: (i, 0)),
          pl.BlockSpec((1, _HID2), lambda i: (0, 0)),
          pl.BlockSpec((_BR, 1), lambda i: (i, 0)),
          pl.BlockSpec((_HID2, _N_DCS), lambda i: (0, 0)),
          pl.BlockSpec((1, _N_DCS), lambda i: (0, 0)),
      ],
      out_specs=pl.BlockSpec((_N_GRAPHS, _N_DCS), lambda i: (0, 0)),
      out_shape=jax.ShapeDtypeStruct((_N_GRAPHS, _N_DCS), jnp.float32),
      scratch_shapes=[pltpu.VMEM((_N_GRAPHS, _N_DCS), jnp.float32)],
  )(p2, h2s, dis, b2, ids, wm, bm)


def kernel(x, edge_index, batch, W1, b1, W2, b2, Wm, bm):
  src = edge_index[0]
  dst = edge_index[1]
  pad = _E_PAD - _E

  def _layout(flat, fill):
    flat = jnp.concatenate([flat, jnp.full((pad,), fill, jnp.int32)])
    p0 = flat[:_NS * _CF * _K].reshape(_NS, _CF, _K)
    p1 = flat[_NS * _CF * _K:].reshape(_NS, _CS, _K)
    p1 = jnp.concatenate(
        [p1, jnp.full((_NS, _CF - _CS, _K), fill, jnp.int32)], axis=1)
    return jnp.concatenate([p0, p1], axis=0).reshape(_NW * _CF, _K)

  srcm = _layout(src, 0)
  dstm = _layout(dst, _N)

  ones_rows = jnp.ones((_K, _DEG_W), jnp.float32)
  z_deg = jnp.zeros((_RPT, _DEG_W), jnp.float32)
  z1 = jnp.zeros((_RPT, _HID), jnp.bfloat16)
  z2 = jnp.zeros((_RPT, _HID2), jnp.bfloat16)

  counts = _sc_deg()(dstm, ones_rows, z_deg)
  h1s, dis = _tc1(counts, x, W1)
  p1 = _sc_agg(_HID)(h1s, srcm, dstm, z1)
  h2s = _tc2(p1, h1s, dis, b1.reshape(1, _HID), W2)
  p2 = _sc_agg(_HID2)(h2s, srcm, dstm, z2)
  return _tc3(p2, h2s, dis, b2.reshape(1, _HID2),
              batch.reshape(_N, 1), Wm, bm.reshape(1, _N_DCS))
